# Initial kernel scaffold; baseline (speedup 1.0000x reference)
import jax
import jax.numpy as jnp
from jax.experimental import pallas as pl


def kernel(z, edge_index, weight1, gru1_wih, gru1_whh, gru1_bih, gru1_bhh, weight2, gru2_wih, gru2_whh, gru2_bih, gru2_bhh, lin_w, lin_b):
    raise NotImplementedError("write your pallas kernel here")


# speedup vs baseline: 8.4281x; 8.4281x over previous
import functools

import jax
import jax.numpy as jnp
from jax import lax
from jax.experimental import pallas as pl
from jax.experimental.pallas import tpu as pltpu
from jax.experimental.pallas import tpu_sc as plsc

N = 10000
E = 320000
D = 128

NC = 2
NS = 16
NW = NC * NS
EPW = E // NW
CHUNK = 80
NCHUNK = EPW // CHUNK
BLK = 25
NBLK = NCHUNK // BLK

NP = 10240
SLAB = NP // NS
ZROWS = 80

RB = 1024


def _sc_scatter_body(m_hbm, src_hbm, dst_hbm, zero_hbm, out_hbm,
                     sv0, dv0, sv1, dv1, rows0, rows1, sem0, sem1, isem,
                     acc):
    cid = lax.axis_index("c")
    sid = lax.axis_index("s")
    wid = cid * NS + sid

    pltpu.sync_copy(zero_hbm, rows0)
    for k in range(SLAB // ZROWS):
        pltpu.sync_copy(rows0, acc.at[pl.ds(sid * SLAB + k * ZROWS, ZROWS)])

    pltpu.sync_copy(src_hbm.at[wid, 0], sv0)
    pltpu.sync_copy(dst_hbm.at[wid, 0], dv0)
    plsc.subcore_barrier()

    ibufs = [(sv0, dv0), (sv1, dv1)]
    for b in range(NBLK):
        sv, dv = ibufs[b % 2]
        svn, dvn = ibufs[(b + 1) % 2]
        if b + 1 < NBLK:
            pltpu.async_copy(src_hbm.at[wid, b + 1], svn, isem)
            pltpu.async_copy(dst_hbm.at[wid, b + 1], dvn, isem)

        def gather(j, buf, sem):
            return pltpu.async_copy(m_hbm.at[sv.at[j]], buf, sem)

        def gwait(j, buf, sem):
            pltpu.make_async_copy(m_hbm.at[sv.at[j]], buf, sem).wait()

        def scatter(j, buf):
            pltpu.sync_copy(buf, acc.at[dv.at[j]], add=True)

        gather(0, rows0, sem0)

        def pair(j, _):
            gather(j + 1, rows1, sem1)
            gwait(j, rows0, sem0)
            scatter(j, rows0)
            gather(j + 2, rows0, sem0)
            gwait(j + 1, rows1, sem1)
            scatter(j + 1, rows1)
            return ()

        lax.fori_loop(0, (BLK - 1) // 2, lambda t, c: pair(2 * t, c), (),
                      unroll=False)
        gwait(BLK - 1, rows0, sem0)
        scatter(BLK - 1, rows0)

        if b + 1 < NBLK:
            pltpu.make_async_copy(src_hbm.at[wid, b + 1], svn, isem).wait()
            pltpu.make_async_copy(dst_hbm.at[wid, b + 1], dvn, isem).wait()

    plsc.subcore_barrier()
    pltpu.sync_copy(acc.at[pl.ds(sid * SLAB, SLAB)],
                    out_hbm.at[cid, pl.ds(sid * SLAB, SLAB)])


def _make_sc_scatter():
    mesh = plsc.VectorSubcoreMesh(core_axis_name="c", subcore_axis_name="s")
    return pl.kernel(
        _sc_scatter_body,
        out_type=jax.ShapeDtypeStruct((NC, NP, D), jnp.float32),
        mesh=mesh,
        scratch_types=[
            pltpu.VMEM((BLK, CHUNK), jnp.int32),
            pltpu.VMEM((BLK, CHUNK), jnp.int32),
            pltpu.VMEM((BLK, CHUNK), jnp.int32),
            pltpu.VMEM((BLK, CHUNK), jnp.int32),
            pltpu.VMEM((CHUNK, D), jnp.float32),
            pltpu.VMEM((CHUNK, D), jnp.float32),
            pltpu.SemaphoreType.DMA,
            pltpu.SemaphoreType.DMA,
            pltpu.SemaphoreType.DMA,
            pltpu.VMEM_SHARED((NP, D), jnp.float32),
        ],
        name="sc_edge_scatter",
    )


def _mm_body(x_ref, w_ref, o_ref):
    o_ref[...] = jnp.dot(x_ref[...], w_ref[0], preferred_element_type=jnp.float32)


def _mm(x, w, j):
    return pl.pallas_call(
        _mm_body,
        grid=(NP // RB,),
        in_specs=[
            pl.BlockSpec((RB, D), lambda i: (i, 0)),
            pl.BlockSpec((1, D, D), lambda i: (j, 0, 0)),
        ],
        out_specs=pl.BlockSpec((RB, D), lambda i: (i, 0)),
        out_shape=jax.ShapeDtypeStruct((NP, D), jnp.float32),
    )(x, w)


def _gru_step_body(apply_tanh, p0_ref, p1_ref, h_ref, wih_ref, whh_ref,
                   bih_ref, bhh_ref, wn_ref, hout_ref, mout_ref):
    agg = p0_ref[0] + p1_ref[0]
    h = h_ref[...]
    gi = lax.dot_general(agg, wih_ref[...], (((1,), (1,)), ((), ())),
                         preferred_element_type=jnp.float32)
    gi = gi + bih_ref[...]
    gh = lax.dot_general(h, whh_ref[...], (((1,), (1,)), ((), ())),
                         preferred_element_type=jnp.float32)
    gh = gh + bhh_ref[...]
    r = jax.nn.sigmoid(gi[:, :D] + gh[:, :D])
    zg = jax.nn.sigmoid(gi[:, D:2 * D] + gh[:, D:2 * D])
    n = jnp.tanh(gi[:, 2 * D:] + r * gh[:, 2 * D:])
    hn = (1.0 - zg) * n + zg * h
    if apply_tanh:
        hn = jnp.tanh(hn)
    hout_ref[...] = hn
    mout_ref[...] = jnp.dot(hn, wn_ref[0], preferred_element_type=jnp.float32)


def _gru_step(parts, h, wih, whh, bih, bhh, wn_src, wn_j, apply_tanh):
    return pl.pallas_call(
        functools.partial(_gru_step_body, apply_tanh),
        grid=(NP // RB,),
        in_specs=[
            pl.BlockSpec((1, RB, D), lambda i: (0, i, 0)),
            pl.BlockSpec((1, RB, D), lambda i: (1, i, 0)),
            pl.BlockSpec((RB, D), lambda i: (i, 0)),
            pl.BlockSpec((3 * D, D), lambda i: (0, 0)),
            pl.BlockSpec((3 * D, D), lambda i: (0, 0)),
            pl.BlockSpec((1, 3 * D), lambda i: (0, 0)),
            pl.BlockSpec((1, 3 * D), lambda i: (0, 0)),
            pl.BlockSpec((1, D, D), lambda i: (wn_j, 0, 0)),
        ],
        out_specs=[
            pl.BlockSpec((RB, D), lambda i: (i, 0)),
            pl.BlockSpec((RB, D), lambda i: (i, 0)),
        ],
        out_shape=[
            jax.ShapeDtypeStruct((NP, D), jnp.float32),
            jax.ShapeDtypeStruct((NP, D), jnp.float32),
        ],
    )(parts, parts, h, wih, whh, bih, bhh, wn_src)


def kernel(z, edge_index, weight1, gru1_wih, gru1_whh, gru1_bih, gru1_bhh,
           weight2, gru2_wih, gru2_whh, gru2_bih, gru2_bhh, lin_w, lin_b):
    dst_flat = edge_index[1].astype(jnp.int32)
    order = jnp.argsort(dst_flat, stable=True)
    src = edge_index[0].astype(jnp.int32)[order].reshape(NW, NBLK, BLK, CHUNK)
    dst = dst_flat[order].reshape(NW, NBLK, BLK, CHUNK)
    zeros = jnp.zeros((ZROWS, D), jnp.float32)
    lin_w_pad = jnp.pad(lin_w, ((0, 0), (0, D - lin_w.shape[1]))).reshape(
        1, D, D)
    z_pad = jnp.pad(z, ((0, NP - N), (0, 0)))

    sc_scatter = _make_sc_scatter()

    L = weight1.shape[0]
    convs = [
        (weight1, gru1_wih, gru1_whh, gru1_bih.reshape(1, -1),
         gru1_bhh.reshape(1, -1)),
        (weight2, gru2_wih, gru2_whh, gru2_bih.reshape(1, -1),
         gru2_bhh.reshape(1, -1)),
    ]

    h = z_pad
    m = _mm(z_pad, weight1, 0)
    for ci, (w, wih, whh, bih, bhh) in enumerate(convs):
        for i in range(L):
            parts = sc_scatter(m, src, dst, zeros)
            last = i == L - 1
            if not last:
                wn_src, wn_j = w, i + 1
            elif ci == 0:
                wn_src, wn_j = weight2, 0
            else:
                wn_src, wn_j = lin_w_pad, 0
            h, m = _gru_step(parts, h, wih, whh, bih, bhh, wn_src, wn_j,
                             apply_tanh=last)
    return m[:N, :1] + lin_b
